# baseline (speedup 1.0000x reference)
import functools

import jax
import jax.numpy as jnp
from jax import lax
from jax.experimental import pallas as pl
from jax.experimental.pallas import tpu as pltpu
from jax.experimental.pallas import tpu_sc as plsc

W = 20
DIM = 64
LANES = 16
ND = DIM // LANES


@functools.cache
def _make_sc_kernel(B):
    info = plsc.get_sparse_core_info()
    nc, ns = info.num_cores, info.num_subcores
    nw = nc * ns
    assert B % nw == 0
    bpw = B // nw
    nt = bpw // LANES
    nph = W * nt

    mesh = plsc.VectorSubcoreMesh(core_axis_name="c", subcore_axis_name="s")

    @functools.partial(
        pl.kernel,
        mesh=mesh,
        out_type=jax.ShapeDtypeStruct((B, DIM), jnp.float32),
        scratch_types=[
            pltpu.VMEM((W, bpw), jnp.int32),
            pltpu.VMEM((W, bpw), jnp.float32),
            pltpu.VMEM((W, bpw), jnp.float32),
            pltpu.VMEM((4, LANES, 8, DIM), jnp.float32),
            pltpu.VMEM((bpw, DIM), jnp.float32),
            pltpu.SemaphoreType.DMA,
            pltpu.SemaphoreType.DMA,
            pltpu.SemaphoreType.DMA,
            pltpu.SemaphoreType.DMA,
        ],
    )
    def k(table, idx, rew, live, out, idx_v, wgt_v, live_v, blk_v, acc_v,
          sem0, sem1, sem2, sem3):
        wid = lax.axis_index("s") * nc + lax.axis_index("c")
        base = wid * bpw
        col = pl.ds(base, bpw)
        sems = (sem0, sem1, sem2, sem3)
        nring = len(sems)

        pltpu.sync_copy(idx.at[:, col], idx_v)

        def fire(p, slot):
            w = p // nt
            b0 = pl.multiple_of((p % nt) * LANES, LANES)
            vec = idx_v[w, pl.ds(b0, LANES)]
            for ll in range(LANES):
                rb = pl.multiple_of((vec[ll] >> 3) << 3, 8)
                pltpu.async_copy(table.at[pl.ds(rb, 8), :],
                                 blk_v.at[slot, ll], sems[slot])

        def drain(slot):
            for _ in range(LANES):
                pltpu.make_async_copy(table.at[pl.ds(0, 8), :],
                                      blk_v.at[0, 0], sems[slot]).wait()

        def accum(p, slot):
            w = p // nt
            b0 = pl.multiple_of((p % nt) * LANES, LANES)
            vec = idx_v[w, pl.ds(b0, LANES)]
            wvec = wgt_v[w, pl.ds(b0, LANES)]
            for ll in range(LANES):
                b = b0 + ll
                srow = vec[ll] & 7
                wv = jnp.full((LANES,), wvec[ll], jnp.float32)
                for d in range(ND):
                    sl = pl.ds(LANES * d, LANES)
                    acc_v[b, sl] = (acc_v[b, sl]
                                    + blk_v[slot, ll, srow, sl] * wv)

        fire(0, 0)
        pltpu.sync_copy(rew.at[:, col], wgt_v)
        pltpu.sync_copy(live.at[:, col], live_v)

        for t in range(nt):
            sl = pl.ds(LANES * t, LANES)
            s = jnp.zeros((LANES,), jnp.float32)
            for w in range(W):
                s = s + live_v[w, sl]
            inv = 1.0 / s
            for w in range(W):
                wgt_v[w, sl] = wgt_v[w, sl] * live_v[w, sl] * inv
            z = jnp.zeros((LANES,), jnp.float32)
            for b in range(LANES):
                for d in range(ND):
                    acc_v[t * LANES + b, pl.ds(LANES * d, LANES)] = z

        for q in range(1, nring):
            fire(q, q)

        def body(g, carry):
            for q in range(nring):
                p = g * nring + q
                drain(q)
                accum(p, q)
                fire(p + nring, q)
            return carry

        lax.fori_loop(0, nph // nring - 1, body, 0)
        for q in range(nring):
            p = nph - nring + q
            drain(q)
            accum(p, q)

        pltpu.sync_copy(acc_v, out.at[pl.ds(base, bpw), :])

    return k


def kernel(item_table, indices, rew, live_mat):
    w, b = live_mat.shape
    assert w == W
    idx2 = indices.reshape(W, b)
    rew2 = rew.reshape(W, b)
    live2 = live_mat.astype(jnp.float32)
    return _make_sc_kernel(b)(item_table, idx2, rew2, live2)


# speedup vs baseline: 1.0129x; 1.0129x over previous
import functools

import jax
import jax.numpy as jnp
from jax import lax
from jax.experimental import pallas as pl
from jax.experimental.pallas import tpu as pltpu
from jax.experimental.pallas import tpu_sc as plsc

W = 20
DIM = 64
LANES = 16
ND = DIM // LANES


@functools.cache
def _make_sc_kernel(B):
    info = plsc.get_sparse_core_info()
    nc, ns = info.num_cores, info.num_subcores
    nw = nc * ns
    assert B % nw == 0
    bpw = B // nw
    nt = bpw // LANES
    nph = W * nt

    mesh = plsc.VectorSubcoreMesh(core_axis_name="c", subcore_axis_name="s")

    @functools.partial(
        pl.kernel,
        mesh=mesh,
        out_type=jax.ShapeDtypeStruct((B, DIM), jnp.float32),
        scratch_types=[
            pltpu.VMEM((W, bpw), jnp.int32),
            pltpu.VMEM((W, bpw), jnp.float32),
            pltpu.VMEM((W, bpw), jnp.float32),
            pltpu.VMEM((5, LANES, 8, DIM), jnp.float32),
            pltpu.VMEM((bpw, DIM), jnp.float32),
            pltpu.SemaphoreType.DMA,
            pltpu.SemaphoreType.DMA,
            pltpu.SemaphoreType.DMA,
            pltpu.SemaphoreType.DMA,
            pltpu.SemaphoreType.DMA,
        ],
    )
    def k(table, idx, rew, live, out, idx_v, wgt_v, live_v, blk_v, acc_v,
          sem0, sem1, sem2, sem3, sem4):
        wid = lax.axis_index("s") * nc + lax.axis_index("c")
        base = wid * bpw
        col = pl.ds(base, bpw)
        sems = (sem0, sem1, sem2, sem3, sem4)
        nring = len(sems)

        pltpu.sync_copy(idx.at[:, col], idx_v)

        def fire(p, slot):
            w = p // nt
            b0 = pl.multiple_of((p % nt) * LANES, LANES)
            vec = idx_v[w, pl.ds(b0, LANES)]
            for ll in range(LANES):
                rb = pl.multiple_of((vec[ll] >> 3) << 3, 8)
                pltpu.async_copy(table.at[pl.ds(rb, 8), :],
                                 blk_v.at[slot, ll], sems[slot])

        def drain(slot):
            pltpu.make_async_copy(table.at[pl.ds(0, LANES * 8), :],
                                  acc_v, sems[slot]).wait()

        def accum(p, slot):
            w = p // nt
            b0 = pl.multiple_of((p % nt) * LANES, LANES)
            vec = idx_v[w, pl.ds(b0, LANES)]
            wvec = wgt_v[w, pl.ds(b0, LANES)]
            for ll in range(LANES):
                b = b0 + ll
                srow = vec[ll] & 7
                wv = jnp.full((LANES,), wvec[ll], jnp.float32)
                for d in range(ND):
                    sl = pl.ds(LANES * d, LANES)
                    acc_v[b, sl] = (acc_v[b, sl]
                                    + blk_v[slot, ll, srow, sl] * wv)

        fire(0, 0)
        pltpu.sync_copy(rew.at[:, col], wgt_v)
        pltpu.sync_copy(live.at[:, col], live_v)

        for t in range(nt):
            sl = pl.ds(LANES * t, LANES)
            s = jnp.zeros((LANES,), jnp.float32)
            for w in range(W):
                s = s + live_v[w, sl]
            inv = 1.0 / s
            for w in range(W):
                wgt_v[w, sl] = wgt_v[w, sl] * live_v[w, sl] * inv
            z = jnp.zeros((LANES,), jnp.float32)
            for b in range(LANES):
                for d in range(ND):
                    acc_v[t * LANES + b, pl.ds(LANES * d, LANES)] = z

        for q in range(1, nring):
            fire(q, q)

        def body(g, carry):
            for q in range(nring):
                p = g * nring + q
                drain(q)
                accum(p, q)
                fire(p + nring, q)
            return carry

        lax.fori_loop(0, nph // nring - 1, body, 0)
        for q in range(nring):
            p = nph - nring + q
            drain(q)
            accum(p, q)

        pltpu.sync_copy(acc_v, out.at[pl.ds(base, bpw), :])

    return k


def kernel(item_table, indices, rew, live_mat):
    w, b = live_mat.shape
    assert w == W
    idx2 = indices.reshape(W, b)
    rew2 = rew.reshape(W, b)
    live2 = live_mat.astype(jnp.float32)
    return _make_sc_kernel(b)(item_table, idx2, rew2, live2)
